# baseline (device time: 19903 ns/iter reference)
import jax
import jax.numpy as jnp
from jax import lax
from jax.experimental import pallas as pl
from jax.experimental.pallas import tpu as pltpu

N_DEV = 8
B = 128
D = 128
H = 256
ROUNDS = 3
ROWS_PER = B // N_DEV

SEND_ORDER = [6, 2, 5, 7, 1, 3, 4]
WAIT_ORDER = [1, 3, 4, 2, 5, 7, 6]


def kernel(x, Win0, Wout0, Win1, Wout1, Win2, Wout2):
    def body(
        wins_hbm, wouts_hbm,
        out_ref, x_vmem, win_vmem, wout_vmem, send_buf, comm_buf, rs_buf,
        local_sems, send_sems, recv_sems,
    ):
        my = lax.axis_index("i")

        cx = pltpu.make_async_copy(
            wins_hbm.at[ROUNDS, :, pl.ds(0, D)], x_vmem, local_sems.at[0]
        )
        cx.start()
        cwin, cwout = [], []
        for k in range(ROUNDS):
            c = pltpu.make_async_copy(wins_hbm.at[k], win_vmem.at[k],
                                      local_sems.at[1 + k])
            c.start()
            cwin.append(c)
            c = pltpu.make_async_copy(wouts_hbm.at[k], wout_vmem.at[k],
                                      local_sems.at[4 + k])
            c.start()
            cwout.append(c)

        barrier = pltpu.get_barrier_semaphore()
        for g in range(1, N_DEV):
            pl.semaphore_signal(
                barrier, inc=1,
                device_id=(my ^ g,), device_id_type=pl.DeviceIdType.MESH,
            )

        cx.wait()
        x_val = x_vmem[:, :]
        for r in range(ROUNDS):
            last = r == ROUNDS - 1
            cwin[r].wait()
            cwout[r].wait()
            h = jnp.maximum(
                jnp.dot(x_val, win_vmem[r],
                        preferred_element_type=jnp.float32),
                0.0,
            ).astype(jnp.bfloat16)
            p = jnp.dot(h, wout_vmem[r],
                        preferred_element_type=jnp.float32)
            send_buf[r, :, :] = p.astype(jnp.bfloat16)

            if r == 0:
                pl.semaphore_wait(barrier, N_DEV - 1)

            rdmas = []
            for g in SEND_ORDER:
                peer = my ^ g
                if last:
                    rdma = pltpu.make_async_remote_copy(
                        src_ref=send_buf.at[r, pl.ds(peer * ROWS_PER, ROWS_PER)],
                        dst_ref=rs_buf.at[g],
                        send_sem=send_sems.at[r, g],
                        recv_sem=recv_sems.at[r, g],
                        device_id=(peer,),
                        device_id_type=pl.DeviceIdType.MESH,
                    )
                else:
                    rdma = pltpu.make_async_remote_copy(
                        src_ref=send_buf.at[r],
                        dst_ref=comm_buf.at[r, g],
                        send_sem=send_sems.at[r, g],
                        recv_sem=recv_sems.at[r, g],
                        device_id=(peer,),
                        device_id_type=pl.DeviceIdType.MESH,
                    )
                rdma.start()
                rdmas.append(rdma)
            for rdma in rdmas:
                rdma.wait_send()

            row0 = my * ROWS_PER
            if last:
                acc = send_buf[r, pl.ds(row0, ROWS_PER), :].astype(jnp.float32)
            else:
                acc = p
            for g in WAIT_ORDER:
                if last:
                    dst_region = rs_buf.at[g]
                    dummy_src = send_buf.at[r, pl.ds(0, ROWS_PER)]
                else:
                    dst_region = comm_buf.at[r, g]
                    dummy_src = send_buf.at[r]
                recv = pltpu.make_async_remote_copy(
                    src_ref=dummy_src,
                    dst_ref=dst_region,
                    send_sem=send_sems.at[r, g],
                    recv_sem=recv_sems.at[r, g],
                    device_id=(my ^ g,),
                    device_id_type=pl.DeviceIdType.MESH,
                )
                recv.wait_recv()
                if last:
                    acc = acc + rs_buf[g].astype(jnp.float32)
                else:
                    acc = acc + comm_buf[r, g].astype(jnp.float32)
            x_val = acc.astype(jnp.bfloat16) if not last else acc

        out_ref[:, :] = x_val

    return pl.pallas_call(
        body,
        out_shape=jax.ShapeDtypeStruct((ROWS_PER, D), jnp.float32),
        in_specs=[pl.BlockSpec(memory_space=pl.ANY)] * 2,
        out_specs=pl.BlockSpec(memory_space=pltpu.VMEM),
        scratch_shapes=[
            pltpu.VMEM((B, D), jnp.bfloat16),
            pltpu.VMEM((ROUNDS, D, H), jnp.bfloat16),
            pltpu.VMEM((ROUNDS, H, D), jnp.bfloat16),
            pltpu.VMEM((ROUNDS, B, D), jnp.bfloat16),
            pltpu.VMEM((ROUNDS - 1, N_DEV, B, D), jnp.bfloat16),
            pltpu.VMEM((N_DEV, ROWS_PER, D), jnp.bfloat16),
            pltpu.SemaphoreType.DMA((7,)),
            pltpu.SemaphoreType.DMA((ROUNDS, N_DEV)),
            pltpu.SemaphoreType.DMA((ROUNDS, N_DEV)),
        ],
        compiler_params=pltpu.CompilerParams(collective_id=0),
    )(
        jnp.stack(
            [Win0.astype(jnp.bfloat16),
             Win1.astype(jnp.bfloat16),
             Win2.astype(jnp.bfloat16),
             jnp.pad(x, ((0, 0), (0, H - D))).astype(jnp.bfloat16)]
        ),
        jnp.stack(
            [Wout0.astype(jnp.bfloat16),
             Wout1.astype(jnp.bfloat16),
             Wout2.astype(jnp.bfloat16)]
        ),
    )


# device time: 18897 ns/iter; 1.0532x vs baseline; 1.0532x over previous
import jax
import jax.numpy as jnp
from jax import lax
from jax.experimental import pallas as pl
from jax.experimental.pallas import tpu as pltpu

N_DEV = 8
B = 128
D = 128
H = 256
ROUNDS = 3
ROWS_PER = B // N_DEV

SEND_ORDER = [6, 2, 5, 7, 1, 3, 4]
WAIT_ORDER = [1, 3, 4, 2, 5, 7, 6]


def kernel(x, Win0, Wout0, Win1, Wout1, Win2, Wout2):
    def body(
        wins_hbm, wouts_hbm,
        out_ref, x_vmem, win_vmem, wout_vmem, send_buf, comm_buf, rs_buf,
        local_sems, send_sems, recv_sems,
    ):
        my = lax.axis_index("i")

        cx = pltpu.make_async_copy(
            wins_hbm.at[ROUNDS, :, pl.ds(0, D)], x_vmem, local_sems.at[0]
        )
        cx.start()
        cwin, cwout = [], []
        for k in range(ROUNDS):
            c = pltpu.make_async_copy(wins_hbm.at[k], win_vmem.at[k],
                                      local_sems.at[1 + k])
            c.start()
            cwin.append(c)
            c = pltpu.make_async_copy(wouts_hbm.at[k], wout_vmem.at[k],
                                      local_sems.at[4 + k])
            c.start()
            cwout.append(c)

        barrier = pltpu.get_barrier_semaphore()
        for g in range(1, N_DEV):
            pl.semaphore_signal(
                barrier, inc=1,
                device_id=(my ^ g,), device_id_type=pl.DeviceIdType.MESH,
            )

        cx.wait()
        x_val = x_vmem[:, :].astype(jnp.bfloat16)
        for r in range(ROUNDS):
            last = r == ROUNDS - 1
            cwin[r].wait()
            cwout[r].wait()
            h = jnp.maximum(
                jnp.dot(x_val, win_vmem[r].astype(jnp.bfloat16),
                        preferred_element_type=jnp.float32),
                0.0,
            ).astype(jnp.bfloat16)
            p = jnp.dot(h, wout_vmem[r].astype(jnp.bfloat16),
                        preferred_element_type=jnp.float32)
            send_buf[r, :, :] = p.astype(jnp.bfloat16)

            if r == 0:
                pl.semaphore_wait(barrier, N_DEV - 1)

            rdmas = []
            for g in SEND_ORDER:
                peer = my ^ g
                if last:
                    rdma = pltpu.make_async_remote_copy(
                        src_ref=send_buf.at[r, pl.ds(peer * ROWS_PER, ROWS_PER)],
                        dst_ref=rs_buf.at[g],
                        send_sem=send_sems.at[r, g],
                        recv_sem=recv_sems.at[r, g],
                        device_id=(peer,),
                        device_id_type=pl.DeviceIdType.MESH,
                    )
                else:
                    rdma = pltpu.make_async_remote_copy(
                        src_ref=send_buf.at[r],
                        dst_ref=comm_buf.at[r, g],
                        send_sem=send_sems.at[r, g],
                        recv_sem=recv_sems.at[r, g],
                        device_id=(peer,),
                        device_id_type=pl.DeviceIdType.MESH,
                    )
                rdma.start()
                rdmas.append(rdma)
            for rdma in rdmas:
                rdma.wait_send()

            row0 = my * ROWS_PER
            if last:
                acc = send_buf[r, pl.ds(row0, ROWS_PER), :].astype(jnp.float32)
            else:
                acc = p
            for g in WAIT_ORDER:
                if last:
                    dst_region = rs_buf.at[g]
                    dummy_src = send_buf.at[r, pl.ds(0, ROWS_PER)]
                else:
                    dst_region = comm_buf.at[r, g]
                    dummy_src = send_buf.at[r]
                recv = pltpu.make_async_remote_copy(
                    src_ref=dummy_src,
                    dst_ref=dst_region,
                    send_sem=send_sems.at[r, g],
                    recv_sem=recv_sems.at[r, g],
                    device_id=(my ^ g,),
                    device_id_type=pl.DeviceIdType.MESH,
                )
                recv.wait_recv()
                if last:
                    acc = acc + rs_buf[g].astype(jnp.float32)
                else:
                    acc = acc + comm_buf[r, g].astype(jnp.float32)
            x_val = acc.astype(jnp.bfloat16) if not last else acc

        out_ref[:, :] = x_val

    return pl.pallas_call(
        body,
        out_shape=jax.ShapeDtypeStruct((ROWS_PER, D), jnp.float32),
        in_specs=[pl.BlockSpec(memory_space=pl.ANY)] * 2,
        out_specs=pl.BlockSpec(memory_space=pltpu.VMEM),
        scratch_shapes=[
            pltpu.VMEM((B, D), jnp.float32),
            pltpu.VMEM((ROUNDS, D, H), jnp.float32),
            pltpu.VMEM((ROUNDS, H, D), jnp.float32),
            pltpu.VMEM((ROUNDS, B, D), jnp.bfloat16),
            pltpu.VMEM((ROUNDS - 1, N_DEV, B, D), jnp.bfloat16),
            pltpu.VMEM((N_DEV, ROWS_PER, D), jnp.bfloat16),
            pltpu.SemaphoreType.DMA((7,)),
            pltpu.SemaphoreType.DMA((ROUNDS, N_DEV)),
            pltpu.SemaphoreType.DMA((ROUNDS, N_DEV)),
        ],
        compiler_params=pltpu.CompilerParams(collective_id=0),
    )(
        jnp.stack([Win0, Win1, Win2,
                   jnp.pad(x, ((0, 0), (0, H - D)))]),
        jnp.stack([Wout0, Wout1, Wout2]),
    )
